# baseline (speedup 1.0000x reference)
import jax
import jax.numpy as jnp
from jax.experimental import pallas as pl


def _pos_kernel(col_ref, row_ref, s_col_ref, s_row_ref, out_ref):
    d = col_ref.shape[1]
    dn = (((0,), (0,)), ((), ()))
    col = jax.lax.dot_general(col_ref[...], s_col_ref[...], dn,
                              preferred_element_type=jnp.float32)
    row = jax.lax.dot_general(row_ref[...], s_row_ref[...], dn,
                              preferred_element_type=jnp.float32)
    out_ref[0, :d, :] = col
    out_ref[0, d:, :] = row


def kernel(x, row_embed, col_embed):
    b = x.shape[0]
    h, w = x.shape[-2], x.shape[-1]
    d = row_embed.shape[1]
    hw = h * w
    k = jnp.arange(hw, dtype=jnp.int32)
    s_col = (k[None, :] % w == jnp.arange(w, dtype=jnp.int32)[:, None]
             ).astype(jnp.float32)
    s_row = (k[None, :] // w == jnp.arange(h, dtype=jnp.int32)[:, None]
             ).astype(jnp.float32)
    out = pl.pallas_call(
        _pos_kernel,
        grid=(b,),
        in_specs=[
            pl.BlockSpec((w, d), lambda i: (0, 0)),
            pl.BlockSpec((h, d), lambda i: (0, 0)),
            pl.BlockSpec((w, hw), lambda i: (0, 0)),
            pl.BlockSpec((h, hw), lambda i: (0, 0)),
        ],
        out_specs=pl.BlockSpec((1, 2 * d, hw), lambda i: (i, 0, 0)),
        out_shape=jax.ShapeDtypeStruct((b, 2 * d, hw), jnp.float32),
    )(col_embed[:w], row_embed[:h], s_col, s_row)
    return out.reshape(b, 2 * d, h, w)


# speedup vs baseline: 1.0377x; 1.0377x over previous
import jax
import jax.numpy as jnp
from jax.experimental import pallas as pl
from jax.experimental.pallas import tpu as pltpu


def _pos_kernel(col_ref, row_ref, s_col_ref, s_row_ref, out_ref, scratch, sem):
    b = out_ref.shape[0]
    d = col_ref.shape[1]
    dn = (((0,), (0,)), ((), ()))
    scratch[:d, :] = jax.lax.dot_general(
        col_ref[...], s_col_ref[...], dn,
        preferred_element_type=jnp.float32,
        precision=jax.lax.Precision.HIGHEST)
    scratch[d:, :] = jax.lax.dot_general(
        row_ref[...], s_row_ref[...], dn,
        preferred_element_type=jnp.float32,
        precision=jax.lax.Precision.HIGHEST)
    copies = [pltpu.make_async_copy(scratch, out_ref.at[i], sem)
              for i in range(b)]
    for c in copies:
        c.start()
    for c in copies:
        c.wait()


def kernel(x, row_embed, col_embed):
    b = x.shape[0]
    h, w = x.shape[-2], x.shape[-1]
    d = row_embed.shape[1]
    hw = h * w
    k = jnp.arange(hw, dtype=jnp.int32)
    s_col = (k[None, :] % w == jnp.arange(w, dtype=jnp.int32)[:, None]
             ).astype(jnp.float32)
    s_row = (k[None, :] // w == jnp.arange(h, dtype=jnp.int32)[:, None]
             ).astype(jnp.float32)
    out = pl.pallas_call(
        _pos_kernel,
        in_specs=[
            pl.BlockSpec(memory_space=pltpu.MemorySpace.VMEM),
            pl.BlockSpec(memory_space=pltpu.MemorySpace.VMEM),
            pl.BlockSpec(memory_space=pltpu.MemorySpace.VMEM),
            pl.BlockSpec(memory_space=pltpu.MemorySpace.VMEM),
        ],
        out_specs=pl.BlockSpec(memory_space=pl.ANY),
        out_shape=jax.ShapeDtypeStruct((b, 2 * d, hw), jnp.float32),
        scratch_shapes=[
            pltpu.VMEM((2 * d, hw), jnp.float32),
            pltpu.SemaphoreType.DMA,
        ],
    )(col_embed[:w], row_embed[:h], s_col, s_row)
    return out.reshape(b, 2 * d, h, w)
